# Initial kernel scaffold; baseline (speedup 1.0000x reference)
import functools

import jax
import jax.numpy as jnp
from jax import lax
from jax.experimental import pallas as pl
from jax.experimental.pallas import tpu as pltpu
from jax.experimental.pallas import tpu_sc as plsc

_NC = 2
_NS = 16
_SCH = 512
_IR = _SCH // 128
_TE = 512
_TN = 1000


def _sc_mesh():
    return plsc.VectorSubcoreMesh(core_axis_name="c", subcore_axis_name="s",
                                  num_cores=_NC, num_subcores=_NS)


def _sc_gather(x, ei3):
    N, D = x.shape
    E = ei3.shape[1] * 128
    n_sc = E // _SCH
    per_tile = n_sc // _NS
    rem = n_sc - per_tile * _NS

    @functools.partial(
        pl.kernel,
        out_type=jax.ShapeDtypeStruct((_NC, E, D), x.dtype),
        mesh=_sc_mesh(),
        scratch_types=[
            pltpu.VMEM((_IR, 128), jnp.int32),
            pltpu.VMEM((_SCH, D), x.dtype),
            pltpu.SemaphoreType.DMA,
        ],
    )
    def gather_kernel(x_hbm, ei_hbm, out_hbm, idx_v, rows_v, sem):
        cid = lax.axis_index("c")
        sid = lax.axis_index("s")

        def do_chunk(sc):
            pltpu.sync_copy(ei_hbm.at[cid, pl.ds(sc * _IR, _IR)], idx_v)
            cps = [
                pltpu.async_copy(x_hbm.at[idx_v.at[j]],
                                 rows_v.at[pl.ds(j * 128, 128)], sem)
                for j in range(_IR)
            ]
            for cp in cps:
                cp.wait()
            pltpu.sync_copy(rows_v, out_hbm.at[cid, pl.ds(sc * _SCH, _SCH)])

        def body(g, carry):
            do_chunk(g * _NS + sid)
            return carry

        lax.fori_loop(0, per_tile, body, 0)
        if rem:
            @pl.when(sid < rem)
            def _():
                do_chunk(per_tile * _NS + sid)

    return gather_kernel(x, ei3)


def _sc_scatter(e1, ei3, zeros_n):
    E, D = e1.shape
    N = zeros_n.shape[0]
    n_sc = E // _SCH
    per_tile = n_sc // _NS
    rem = n_sc - per_tile * _NS
    npt = N // _NS

    @functools.partial(
        pl.kernel,
        out_type=jax.ShapeDtypeStruct((_NC, N, D), jnp.float32),
        mesh=_sc_mesh(),
        scratch_types=[
            pltpu.VMEM((_IR, 128), jnp.int32),
            pltpu.VMEM((_SCH, D), jnp.float32),
            pltpu.VMEM_SHARED((N, D), jnp.float32),
        ],
    )
    def scatter_kernel(e1_hbm, ei_hbm, z_hbm, out_hbm, idx_v, rows_v, accum):
        cid = lax.axis_index("c")
        sid = lax.axis_index("s")
        pltpu.sync_copy(z_hbm.at[pl.ds(sid * npt, npt)],
                        accum.at[pl.ds(sid * npt, npt)])
        plsc.subcore_barrier()

        def do_chunk(sc):
            pltpu.sync_copy(ei_hbm.at[cid, pl.ds(sc * _IR, _IR)], idx_v)
            pltpu.sync_copy(e1_hbm.at[pl.ds(sc * _SCH, _SCH)], rows_v)
            for j in range(_IR):
                pltpu.sync_copy(rows_v.at[pl.ds(j * 128, 128)],
                                accum.at[idx_v.at[j]], add=True)

        def body(g, carry):
            do_chunk(g * _NS + sid)
            return carry

        lax.fori_loop(0, per_tile, body, 0)
        if rem:
            @pl.when(sid < rem)
            def _():
                do_chunk(per_tile * _NS + sid)
        plsc.subcore_barrier()
        pltpu.sync_copy(accum.at[pl.ds(sid * npt, npt)],
                        out_hbm.at[cid, pl.ds(sid * npt, npt)])

    return scatter_kernel(e1, ei3, zeros_n)


def _tc_edge(xsxd, ea, eb3, u, W1, W2, W3, b1, b2, b3):
    E, D = ea.shape
    B = u.shape[0]
    G = E // _TE

    def body(xsxd_ref, ea_ref, eb_ref, u_ref, W1_ref, W2_ref, W3_ref,
             b1_ref, b2_ref, b3_ref, e1_ref, eao_ref, agge_ref, ue_scr):
        i = pl.program_id(0)

        @pl.when(i == 0)
        def _():
            ue_scr[...] = jnp.dot(u_ref[...], W1_ref[3 * D:4 * D, :],
                                  preferred_element_type=jnp.float32)
            agge_ref[...] = jnp.zeros_like(agge_ref)

        xs = xsxd_ref[0]
        xd = xsxd_ref[1]
        ea_v = ea_ref[...]
        onehot = (lax.broadcasted_iota(jnp.int32, (B, _TE), 0)
                  == eb_ref[0]).astype(jnp.float32)
        h = (jnp.dot(xs, W1_ref[0:D], preferred_element_type=jnp.float32)
             + jnp.dot(xd, W1_ref[D:2 * D], preferred_element_type=jnp.float32)
             + jnp.dot(ea_v, W1_ref[2 * D:3 * D],
                       preferred_element_type=jnp.float32)
             + lax.dot_general(onehot, ue_scr[...], (((0,), (0,)), ((), ())),
                               preferred_element_type=jnp.float32)
             + b1_ref[...])
        h = jnp.maximum(h, 0.0)
        h = jnp.maximum(
            jnp.dot(h, W2_ref[...], preferred_element_type=jnp.float32)
            + b2_ref[...], 0.0)
        e1 = jnp.maximum(
            jnp.dot(h, W3_ref[...], preferred_element_type=jnp.float32)
            + b3_ref[...], 0.0)
        e1_ref[...] = e1
        eao_ref[...] = e1 + ea_v
        agge_ref[...] += lax.dot_general(onehot, e1, (((1,), (0,)), ((), ())),
                                         preferred_element_type=jnp.float32)

    H = W2.shape[0]
    return pl.pallas_call(
        body,
        grid=(G,),
        in_specs=[
            pl.BlockSpec((2, _TE, D), lambda i: (0, i, 0)),
            pl.BlockSpec((_TE, D), lambda i: (i, 0)),
            pl.BlockSpec((1, 1, _TE), lambda i: (i, 0, 0)),
            pl.BlockSpec((B, D), lambda i: (0, 0)),
            pl.BlockSpec((4 * D, H), lambda i: (0, 0)),
            pl.BlockSpec((H, H), lambda i: (0, 0)),
            pl.BlockSpec((H, D), lambda i: (0, 0)),
            pl.BlockSpec((1, H), lambda i: (0, 0)),
            pl.BlockSpec((1, H), lambda i: (0, 0)),
            pl.BlockSpec((1, D), lambda i: (0, 0)),
        ],
        out_specs=[
            pl.BlockSpec((_TE, D), lambda i: (i, 0)),
            pl.BlockSpec((_TE, D), lambda i: (i, 0)),
            pl.BlockSpec((B, D), lambda i: (0, 0)),
        ],
        out_shape=[
            jax.ShapeDtypeStruct((E, D), jnp.float32),
            jax.ShapeDtypeStruct((E, D), jnp.float32),
            jax.ShapeDtypeStruct((B, D), jnp.float32),
        ],
        scratch_shapes=[pltpu.VMEM((B, H), jnp.float32)],
    )(xsxd, ea, eb3, u, W1, W2, W3, b1, b2, b3)


def _tc_node(sr, x, nb3, u, agg_e, node_params, globl_params):
    N, D = x.shape
    B = u.shape[0]
    G = N // _TN
    (Wn1, bn1), (Wn2, bn2), (Wn3, bn3) = node_params
    has_global = globl_params is not None
    if has_global:
        (Wg1, bg1), (Wg2, bg2), (Wg3, bg3) = globl_params
    H = Wn2.shape[0]

    def body(sr_ref, x_ref, nb_ref, u_ref, agge_ref,
             Wn1_ref, Wn2_ref, Wn3_ref, bn1_ref, bn2_ref, bn3_ref,
             *rest):
        if has_global:
            (Wg1_ref, Wg2_ref, Wg3_ref, bg1_ref, bg2_ref, bg3_ref,
             xo_ref, uo_ref, un_scr, aggn_scr) = rest
        else:
            (xo_ref, un_scr, aggn_scr) = rest
        i = pl.program_id(0)

        @pl.when(i == 0)
        def _():
            un_scr[...] = jnp.dot(u_ref[...], Wn1_ref[3 * D:4 * D, :],
                                  preferred_element_type=jnp.float32)
            aggn_scr[...] = jnp.zeros_like(aggn_scr)

        sent = sr_ref[0]
        recv = sr_ref[1]
        x_v = x_ref[...]
        onehot = (lax.broadcasted_iota(jnp.int32, (B, _TN), 0)
                  == nb_ref[0]).astype(jnp.float32)
        h = (jnp.dot(x_v, Wn1_ref[0:D], preferred_element_type=jnp.float32)
             + jnp.dot(recv, Wn1_ref[D:2 * D],
                       preferred_element_type=jnp.float32)
             + jnp.dot(sent, Wn1_ref[2 * D:3 * D],
                       preferred_element_type=jnp.float32)
             + lax.dot_general(onehot, un_scr[...], (((0,), (0,)), ((), ())),
                               preferred_element_type=jnp.float32)
             + bn1_ref[...])
        h = jnp.maximum(h, 0.0)
        h = jnp.maximum(
            jnp.dot(h, Wn2_ref[...], preferred_element_type=jnp.float32)
            + bn2_ref[...], 0.0)
        x1 = jnp.maximum(
            jnp.dot(h, Wn3_ref[...], preferred_element_type=jnp.float32)
            + bn3_ref[...], 0.0)
        xo_ref[...] = x1 + x_v
        aggn_scr[...] += lax.dot_general(onehot, x1, (((1,), (0,)), ((), ())),
                                         preferred_element_type=jnp.float32)

        if has_global:
            @pl.when(i == G - 1)
            def _():
                u_v = u_ref[...]
                g = jnp.maximum(
                    jnp.dot(u_v, Wg1_ref[0:D],
                            preferred_element_type=jnp.float32)
                    + jnp.dot(aggn_scr[...], Wg1_ref[D:2 * D],
                              preferred_element_type=jnp.float32)
                    + jnp.dot(agge_ref[...], Wg1_ref[2 * D:3 * D],
                              preferred_element_type=jnp.float32)
                    + bg1_ref[...], 0.0)
                g = jnp.maximum(
                    jnp.dot(g, Wg2_ref[...], preferred_element_type=jnp.float32)
                    + bg2_ref[...], 0.0)
                u1 = jnp.maximum(
                    jnp.dot(g, Wg3_ref[...], preferred_element_type=jnp.float32)
                    + bg3_ref[...], 0.0)
                uo_ref[...] = u1 + u_v

    in_specs = [
        pl.BlockSpec((2, _TN, D), lambda i: (0, i, 0)),
        pl.BlockSpec((_TN, D), lambda i: (i, 0)),
        pl.BlockSpec((1, 1, _TN), lambda i: (i, 0, 0)),
        pl.BlockSpec((B, D), lambda i: (0, 0)),
        pl.BlockSpec((B, D), lambda i: (0, 0)),
        pl.BlockSpec((4 * D, H), lambda i: (0, 0)),
        pl.BlockSpec((H, H), lambda i: (0, 0)),
        pl.BlockSpec((H, D), lambda i: (0, 0)),
        pl.BlockSpec((1, H), lambda i: (0, 0)),
        pl.BlockSpec((1, H), lambda i: (0, 0)),
        pl.BlockSpec((1, D), lambda i: (0, 0)),
    ]
    args = [sr, x, nb3, u, agg_e, Wn1, Wn2, Wn3,
            bn1.reshape(1, -1), bn2.reshape(1, -1), bn3.reshape(1, -1)]
    out_specs = [pl.BlockSpec((_TN, D), lambda i: (i, 0))]
    out_shape = [jax.ShapeDtypeStruct((N, D), jnp.float32)]
    if has_global:
        in_specs += [
            pl.BlockSpec((3 * D, H), lambda i: (0, 0)),
            pl.BlockSpec((H, H), lambda i: (0, 0)),
            pl.BlockSpec((H, D), lambda i: (0, 0)),
            pl.BlockSpec((1, H), lambda i: (0, 0)),
            pl.BlockSpec((1, H), lambda i: (0, 0)),
            pl.BlockSpec((1, D), lambda i: (0, 0)),
        ]
        args += [Wg1, Wg2, Wg3,
                 bg1.reshape(1, -1), bg2.reshape(1, -1), bg3.reshape(1, -1)]
        out_specs += [pl.BlockSpec((B, D), lambda i: (0, 0))]
        out_shape += [jax.ShapeDtypeStruct((B, D), jnp.float32)]

    outs = pl.pallas_call(
        body,
        grid=(G,),
        in_specs=in_specs,
        out_specs=out_specs,
        out_shape=out_shape,
        scratch_shapes=[pltpu.VMEM((B, H), jnp.float32),
                        pltpu.VMEM((B, D), jnp.float32)],
    )(*args)
    if has_global:
        return outs[0], outs[1]
    return outs[0], u


def kernel(x, edge_index, edge_attr, u, node_batch, edge_batch, params):
    N, D = x.shape
    E = edge_attr.shape[0]
    ei3 = edge_index.astype(jnp.int32).reshape(2, E // 128, 128)
    eb3 = edge_batch.astype(jnp.int32).reshape(E // _TE, 1, _TE)
    nb3 = node_batch.astype(jnp.int32).reshape(N // _TN, 1, _TN)
    zeros_n = jnp.zeros((N, D), jnp.float32)

    for layer in params:
        (W1, b1), (W2, b2), (W3, b3) = layer['edge']
        xsxd = _sc_gather(x, ei3)
        e1, ea_new, agg_e = _tc_edge(
            xsxd, edge_attr, eb3, u, W1, W2, W3,
            b1.reshape(1, -1), b2.reshape(1, -1), b3.reshape(1, -1))
        sr = _sc_scatter(e1, ei3, zeros_n)
        x, u = _tc_node(sr, x, nb3, u, agg_e, layer['node'], layer['globl'])
        edge_attr = ea_new
    return (x, edge_attr, u)


# speedup vs baseline: 2.5601x; 2.5601x over previous
import functools

import jax
import jax.numpy as jnp
from jax import lax
from jax.experimental import pallas as pl
from jax.experimental.pallas import tpu as pltpu
from jax.experimental.pallas import tpu_sc as plsc

_NC = 2
_NS = 16
_SCH_G = 512
_SCH_S = 256
_TE = 512
_TN = 1000


def _sc_mesh():
    return plsc.VectorSubcoreMesh(core_axis_name="c", subcore_axis_name="s",
                                  num_cores=_NC, num_subcores=_NS)


def _sc_gather(x, ei3):
    N, D = x.shape
    E = ei3.shape[1] * 128
    n_sc = E // _SCH_G
    ir = _SCH_G // 128
    per_tile = n_sc // _NS
    rem = n_sc - per_tile * _NS

    @functools.partial(
        pl.kernel,
        out_type=jax.ShapeDtypeStruct((_NC, E, D), x.dtype),
        mesh=_sc_mesh(),
        scratch_types=[
            pltpu.VMEM((ir, 128), jnp.int32),
            pltpu.VMEM((_SCH_G, D), x.dtype),
            pltpu.SemaphoreType.DMA,
        ],
    )
    def gather_kernel(x_hbm, ei_hbm, out_hbm, idx_v, rows_v, sem):
        cid = lax.axis_index("c")
        sid = lax.axis_index("s")

        def do_chunk(sc):
            pltpu.sync_copy(ei_hbm.at[cid, pl.ds(sc * ir, ir)], idx_v)
            cps = [
                pltpu.async_copy(x_hbm.at[idx_v.at[j]],
                                 rows_v.at[pl.ds(j * 128, 128)], sem)
                for j in range(ir)
            ]
            for cp in cps:
                cp.wait()
            pltpu.sync_copy(rows_v, out_hbm.at[cid, pl.ds(sc * _SCH_G, _SCH_G)])

        def body(g, carry):
            do_chunk(g * _NS + sid)
            return carry

        lax.fori_loop(0, per_tile, body, 0)
        if rem:
            @pl.when(sid < rem)
            def _():
                do_chunk(per_tile * _NS + sid)

    return gather_kernel(x, ei3)


def _sc_scatter(e1, ei3, zeros_n):
    E, D = e1.shape
    N = zeros_n.shape[0]
    n_sc = E // _SCH_S
    ir = _SCH_S // 128
    per_tile = n_sc // _NS
    rem = n_sc - per_tile * _NS
    npt = (N // _NS) // 8 * 8
    ntail = N - npt * _NS

    @functools.partial(
        pl.kernel,
        out_type=jax.ShapeDtypeStruct((_NC, N, D), jnp.float32),
        mesh=_sc_mesh(),
        scratch_types=(
            [pltpu.VMEM((128,), jnp.int32) for _ in range(ir)]
            + [pltpu.VMEM((_SCH_S, D), jnp.float32),
               pltpu.VMEM_SHARED((N, D), jnp.float32)]
        ),
    )
    def scatter_kernel(e1_hbm, ei_hbm, z_hbm, out_hbm, *refs):
        idx_vs = refs[:ir]
        rows_v, accum = refs[ir], refs[ir + 1]
        cid = lax.axis_index("c")
        sid = lax.axis_index("s")
        pltpu.sync_copy(z_hbm.at[pl.ds(sid * npt, npt)],
                        accum.at[pl.ds(sid * npt, npt)])
        if ntail:
            @pl.when(sid == 0)
            def _():
                pltpu.sync_copy(z_hbm.at[pl.ds(npt * _NS, ntail)],
                                accum.at[pl.ds(npt * _NS, ntail)])
        plsc.subcore_barrier()

        def do_chunk(sc):
            for j in range(ir):
                pltpu.sync_copy(ei_hbm.at[cid, sc * ir + j], idx_vs[j])
            pltpu.sync_copy(e1_hbm.at[pl.ds(sc * _SCH_S, _SCH_S)], rows_v)
            for j in range(ir):
                pltpu.sync_copy(rows_v.at[pl.ds(j * 128, 128)],
                                accum.at[idx_vs[j]], add=True)

        def body(g, carry):
            do_chunk(g * _NS + sid)
            return carry

        lax.fori_loop(0, per_tile, body, 0)
        if rem:
            @pl.when(sid < rem)
            def _():
                do_chunk(per_tile * _NS + sid)
        plsc.subcore_barrier()
        pltpu.sync_copy(accum.at[pl.ds(sid * npt, npt)],
                        out_hbm.at[cid, pl.ds(sid * npt, npt)])
        if ntail:
            @pl.when(sid == 0)
            def _():
                pltpu.sync_copy(accum.at[pl.ds(npt * _NS, ntail)],
                                out_hbm.at[cid, pl.ds(npt * _NS, ntail)])

    return scatter_kernel(e1, ei3, zeros_n)


def _tc_edge(xsxd, ea, eb3, u, W1, W2, W3, b1, b2, b3):
    E, D = ea.shape
    B = u.shape[0]
    G = E // _TE

    def body(xsxd_ref, ea_ref, eb_ref, u_ref, W1_ref, W2_ref, W3_ref,
             b1_ref, b2_ref, b3_ref, e1_ref, eao_ref, agge_ref):
        i = pl.program_id(0)

        @pl.when(i == 0)
        def _():
            agge_ref[...] = jnp.zeros_like(agge_ref)

        xs = xsxd_ref[0]
        xd = xsxd_ref[1]
        ea_v = ea_ref[...]
        onehot = (lax.broadcasted_iota(jnp.int32, (B, _TE), 0)
                  == eb_ref[0]).astype(jnp.float32)
        uexp = lax.dot_general(onehot, u_ref[...], (((0,), (0,)), ((), ())),
                               preferred_element_type=jnp.float32,
                               precision=lax.Precision.HIGHEST)
        e_in = jnp.concatenate([xs, xd, ea_v, uexp], axis=1)
        h = jnp.maximum(
            jnp.dot(e_in, W1_ref[...], preferred_element_type=jnp.float32)
            + b1_ref[...], 0.0)
        h = jnp.maximum(
            jnp.dot(h, W2_ref[...], preferred_element_type=jnp.float32)
            + b2_ref[...], 0.0)
        e1 = jnp.maximum(
            jnp.dot(h, W3_ref[...], preferred_element_type=jnp.float32)
            + b3_ref[...], 0.0)
        e1_ref[...] = e1
        eao_ref[...] = e1 + ea_v
        agge_ref[...] += lax.dot_general(onehot, e1, (((1,), (0,)), ((), ())),
                                         preferred_element_type=jnp.float32,
                                         precision=lax.Precision.HIGHEST)

    H = W2.shape[0]
    return pl.pallas_call(
        body,
        grid=(G,),
        in_specs=[
            pl.BlockSpec((2, _TE, D), lambda i: (0, i, 0)),
            pl.BlockSpec((_TE, D), lambda i: (i, 0)),
            pl.BlockSpec((1, 1, _TE), lambda i: (i, 0, 0)),
            pl.BlockSpec((B, D), lambda i: (0, 0)),
            pl.BlockSpec((4 * D, H), lambda i: (0, 0)),
            pl.BlockSpec((H, H), lambda i: (0, 0)),
            pl.BlockSpec((H, D), lambda i: (0, 0)),
            pl.BlockSpec((1, H), lambda i: (0, 0)),
            pl.BlockSpec((1, H), lambda i: (0, 0)),
            pl.BlockSpec((1, D), lambda i: (0, 0)),
        ],
        out_specs=[
            pl.BlockSpec((_TE, D), lambda i: (i, 0)),
            pl.BlockSpec((_TE, D), lambda i: (i, 0)),
            pl.BlockSpec((B, D), lambda i: (0, 0)),
        ],
        out_shape=[
            jax.ShapeDtypeStruct((E, D), jnp.float32),
            jax.ShapeDtypeStruct((E, D), jnp.float32),
            jax.ShapeDtypeStruct((B, D), jnp.float32),
        ],
    )(xsxd, ea, eb3, u, W1, W2, W3, b1, b2, b3)


def _tc_node(sr, x, nb3, u, agg_e, node_params, globl_params):
    N, D = x.shape
    B = u.shape[0]
    G = N // _TN
    (Wn1, bn1), (Wn2, bn2), (Wn3, bn3) = node_params
    has_global = globl_params is not None
    if has_global:
        (Wg1, bg1), (Wg2, bg2), (Wg3, bg3) = globl_params
    H = Wn2.shape[0]

    def body(sr_ref, x_ref, nb_ref, u_ref, agge_ref,
             Wn1_ref, Wn2_ref, Wn3_ref, bn1_ref, bn2_ref, bn3_ref,
             *rest):
        if has_global:
            (Wg1_ref, Wg2_ref, Wg3_ref, bg1_ref, bg2_ref, bg3_ref,
             xo_ref, uo_ref, aggn_scr) = rest
        else:
            (xo_ref, aggn_scr) = rest
        i = pl.program_id(0)

        @pl.when(i == 0)
        def _():
            aggn_scr[...] = jnp.zeros_like(aggn_scr)

        sent = sr_ref[0]
        recv = sr_ref[1]
        x_v = x_ref[...]
        onehot = (lax.broadcasted_iota(jnp.int32, (B, _TN), 0)
                  == nb_ref[0]).astype(jnp.float32)
        unexp = lax.dot_general(onehot, u_ref[...], (((0,), (0,)), ((), ())),
                                preferred_element_type=jnp.float32,
                                precision=lax.Precision.HIGHEST)
        n_in = jnp.concatenate([x_v, recv, sent, unexp], axis=1)
        h = jnp.maximum(
            jnp.dot(n_in, Wn1_ref[...], preferred_element_type=jnp.float32)
            + bn1_ref[...], 0.0)
        h = jnp.maximum(
            jnp.dot(h, Wn2_ref[...], preferred_element_type=jnp.float32)
            + bn2_ref[...], 0.0)
        x1 = jnp.maximum(
            jnp.dot(h, Wn3_ref[...], preferred_element_type=jnp.float32)
            + bn3_ref[...], 0.0)
        xo_ref[...] = x1 + x_v
        aggn_scr[...] += lax.dot_general(onehot, x1, (((1,), (0,)), ((), ())),
                                         preferred_element_type=jnp.float32,
                                         precision=lax.Precision.HIGHEST)

        if has_global:
            @pl.when(i == G - 1)
            def _():
                u_v = u_ref[...]
                g_in = jnp.concatenate(
                    [u_v, aggn_scr[...], agge_ref[...]], axis=1)
                g = jnp.maximum(
                    jnp.dot(g_in, Wg1_ref[...],
                            preferred_element_type=jnp.float32)
                    + bg1_ref[...], 0.0)
                g = jnp.maximum(
                    jnp.dot(g, Wg2_ref[...], preferred_element_type=jnp.float32)
                    + bg2_ref[...], 0.0)
                u1 = jnp.maximum(
                    jnp.dot(g, Wg3_ref[...], preferred_element_type=jnp.float32)
                    + bg3_ref[...], 0.0)
                uo_ref[...] = u1 + u_v

    in_specs = [
        pl.BlockSpec((2, _TN, D), lambda i: (0, i, 0)),
        pl.BlockSpec((_TN, D), lambda i: (i, 0)),
        pl.BlockSpec((1, 1, _TN), lambda i: (i, 0, 0)),
        pl.BlockSpec((B, D), lambda i: (0, 0)),
        pl.BlockSpec((B, D), lambda i: (0, 0)),
        pl.BlockSpec((4 * D, H), lambda i: (0, 0)),
        pl.BlockSpec((H, H), lambda i: (0, 0)),
        pl.BlockSpec((H, D), lambda i: (0, 0)),
        pl.BlockSpec((1, H), lambda i: (0, 0)),
        pl.BlockSpec((1, H), lambda i: (0, 0)),
        pl.BlockSpec((1, D), lambda i: (0, 0)),
    ]
    args = [sr, x, nb3, u, agg_e, Wn1, Wn2, Wn3,
            bn1.reshape(1, -1), bn2.reshape(1, -1), bn3.reshape(1, -1)]
    out_specs = [pl.BlockSpec((_TN, D), lambda i: (i, 0))]
    out_shape = [jax.ShapeDtypeStruct((N, D), jnp.float32)]
    if has_global:
        in_specs += [
            pl.BlockSpec((3 * D, H), lambda i: (0, 0)),
            pl.BlockSpec((H, H), lambda i: (0, 0)),
            pl.BlockSpec((H, D), lambda i: (0, 0)),
            pl.BlockSpec((1, H), lambda i: (0, 0)),
            pl.BlockSpec((1, H), lambda i: (0, 0)),
            pl.BlockSpec((1, D), lambda i: (0, 0)),
        ]
        args += [Wg1, Wg2, Wg3,
                 bg1.reshape(1, -1), bg2.reshape(1, -1), bg3.reshape(1, -1)]
        out_specs += [pl.BlockSpec((B, D), lambda i: (0, 0))]
        out_shape += [jax.ShapeDtypeStruct((B, D), jnp.float32)]

    outs = pl.pallas_call(
        body,
        grid=(G,),
        in_specs=in_specs,
        out_specs=out_specs,
        out_shape=out_shape,
        scratch_shapes=[pltpu.VMEM((B, D), jnp.float32)],
    )(*args)
    if has_global:
        return outs[0], outs[1]
    return outs[0], u


def kernel(x, edge_index, edge_attr, u, node_batch, edge_batch, params):
    N, D = x.shape
    E = edge_attr.shape[0]
    ei3 = edge_index.astype(jnp.int32).reshape(2, E // 128, 128)
    eb3 = edge_batch.astype(jnp.int32).reshape(E // _TE, 1, _TE)
    nb3 = node_batch.astype(jnp.int32).reshape(N // _TN, 1, _TN)
    zeros_n = jnp.zeros((N, D), jnp.float32)

    for layer in params:
        (W1, b1), (W2, b2), (W3, b3) = layer['edge']
        xsxd = _sc_gather(x, ei3)
        e1, ea_new, agg_e = _tc_edge(
            xsxd, edge_attr, eb3, u, W1, W2, W3,
            b1.reshape(1, -1), b2.reshape(1, -1), b3.reshape(1, -1))
        sr = _sc_scatter(e1, ei3, zeros_n)
        x, u = _tc_node(sr, x, nb3, u, agg_e, layer['node'], layer['globl'])
        edge_attr = ea_new
    return (x, edge_attr, u)


# speedup vs baseline: 2.8222x; 1.1024x over previous
import functools

import jax
import jax.numpy as jnp
from jax import lax
from jax.experimental import pallas as pl
from jax.experimental.pallas import tpu as pltpu
from jax.experimental.pallas import tpu_sc as plsc

_NC = 2
_NS = 16
_SCH_G = 512
_SCH_S = 128
_TE = 512
_TN = 1000


def _sc_mesh():
    return plsc.VectorSubcoreMesh(core_axis_name="c", subcore_axis_name="s",
                                  num_cores=_NC, num_subcores=_NS)


def _sc_gather(x, ei3):
    N, D = x.shape
    E = ei3.shape[1] * 128
    n_sc = E // _SCH_G
    ir = _SCH_G // 128
    per_tile = n_sc // _NS
    rem = n_sc - per_tile * _NS

    @functools.partial(
        pl.kernel,
        out_type=jax.ShapeDtypeStruct((_NC, E, D), x.dtype),
        mesh=_sc_mesh(),
        scratch_types=[
            pltpu.VMEM((ir, 128), jnp.int32),
            pltpu.VMEM((_SCH_G, D), x.dtype),
            pltpu.SemaphoreType.DMA,
        ],
    )
    def gather_kernel(x_hbm, ei_hbm, out_hbm, idx_v, rows_v, sem):
        cid = lax.axis_index("c")
        sid = lax.axis_index("s")

        def do_chunk(sc):
            pltpu.sync_copy(ei_hbm.at[cid, pl.ds(sc * ir, ir)], idx_v)
            cps = [
                pltpu.async_copy(x_hbm.at[idx_v.at[j]],
                                 rows_v.at[pl.ds(j * 128, 128)], sem)
                for j in range(ir)
            ]
            for cp in cps:
                cp.wait()
            pltpu.sync_copy(rows_v, out_hbm.at[cid, pl.ds(sc * _SCH_G, _SCH_G)])

        def body(g, carry):
            do_chunk(g * _NS + sid)
            return carry

        lax.fori_loop(0, per_tile, body, 0)
        if rem:
            @pl.when(sid < rem)
            def _():
                do_chunk(per_tile * _NS + sid)

    return gather_kernel(x, ei3)


def _sc_scatter(e1, ei3, zeros_n):
    E, D = e1.shape
    N = zeros_n.shape[0]
    n_sc = E // _SCH_S
    ir = _SCH_S // 128
    per_tile = n_sc // _NS
    rem = n_sc - per_tile * _NS
    npt = (N // _NS) // 8 * 8
    ntail = N - npt * _NS

    @functools.partial(
        pl.kernel,
        out_type=jax.ShapeDtypeStruct((_NC, N, D), jnp.float32),
        mesh=_sc_mesh(),
        scratch_types=(
            [pltpu.VMEM((128,), jnp.int32) for _ in range(2 * ir)]
            + [pltpu.VMEM((_SCH_S, D), jnp.float32),
               pltpu.VMEM((_SCH_S, D), jnp.float32),
               pltpu.VMEM_SHARED((N, D), jnp.float32),
               pltpu.SemaphoreType.DMA,
               pltpu.SemaphoreType.DMA]
        ),
    )
    def scatter_kernel(e1_hbm, ei_hbm, z_hbm, out_hbm, *refs):
        idx_b = (refs[:ir], refs[ir:2 * ir])
        rows_b = (refs[2 * ir], refs[2 * ir + 1])
        accum = refs[2 * ir + 2]
        lsem = (refs[2 * ir + 3], refs[2 * ir + 4])
        cid = lax.axis_index("c")
        sid = lax.axis_index("s")
        pltpu.sync_copy(z_hbm.at[pl.ds(sid * npt, npt)],
                        accum.at[pl.ds(sid * npt, npt)])
        if ntail:
            @pl.when(sid == 0)
            def _():
                pltpu.sync_copy(z_hbm.at[pl.ds(npt * _NS, ntail)],
                                accum.at[pl.ds(npt * _NS, ntail)])
        plsc.subcore_barrier()

        last = per_tile - 1

        def fire(b, sc):
            pltpu.async_copy(e1_hbm.at[pl.ds(sc * _SCH_S, _SCH_S)],
                             rows_b[b], lsem[b])

        def proc(b, sc):
            for j in range(ir):
                pltpu.sync_copy(ei_hbm.at[cid, sc * ir + j], idx_b[b][j])
            pltpu.make_async_copy(e1_hbm.at[pl.ds(sc * _SCH_S, _SCH_S)],
                                  rows_b[b], lsem[b]).wait()
            for j in range(ir):
                pltpu.sync_copy(rows_b[b].at[pl.ds(j * 128, 128)],
                                accum.at[idx_b[b][j]], add=True)

        def chunk(i):
            return i * _NS + sid

        fire(0, chunk(0))
        fire(1, chunk(1))

        def body(s, carry):
            for k in (0, 1):
                i = 2 * s + k
                proc(k, chunk(i))
                nxt = jnp.minimum(i + 2, last)
                fire(k, chunk(nxt))
            return carry

        lax.fori_loop(0, per_tile // 2, body, 0)
        for b in (0, 1):
            pltpu.make_async_copy(e1_hbm.at[pl.ds(0, _SCH_S)],
                                  rows_b[b], lsem[b]).wait()

        def do_chunk(sc):
            for j in range(ir):
                pltpu.sync_copy(ei_hbm.at[cid, sc * ir + j], idx_b[0][j])
            pltpu.sync_copy(e1_hbm.at[pl.ds(sc * _SCH_S, _SCH_S)], rows_b[0])
            for j in range(ir):
                pltpu.sync_copy(rows_b[0].at[pl.ds(j * 128, 128)],
                                accum.at[idx_b[0][j]], add=True)

        if rem:
            @pl.when(sid < rem)
            def _():
                do_chunk(per_tile * _NS + sid)
        plsc.subcore_barrier()
        pltpu.sync_copy(accum.at[pl.ds(sid * npt, npt)],
                        out_hbm.at[cid, pl.ds(sid * npt, npt)])
        if ntail:
            @pl.when(sid == 0)
            def _():
                pltpu.sync_copy(accum.at[pl.ds(npt * _NS, ntail)],
                                out_hbm.at[cid, pl.ds(npt * _NS, ntail)])

    return scatter_kernel(e1, ei3, zeros_n)


def _tc_edge(xsxd, ea, eb3, u, W1, W2, W3, b1, b2, b3):
    E, D = ea.shape
    B = u.shape[0]
    G = E // _TE

    def body(xsxd_ref, ea_ref, eb_ref, u_ref, W1_ref, W2_ref, W3_ref,
             b1_ref, b2_ref, b3_ref, e1_ref, eao_ref, agge_ref):
        i = pl.program_id(0)

        @pl.when(i == 0)
        def _():
            agge_ref[...] = jnp.zeros_like(agge_ref)

        xs = xsxd_ref[0]
        xd = xsxd_ref[1]
        ea_v = ea_ref[...]
        onehot = (lax.broadcasted_iota(jnp.int32, (B, _TE), 0)
                  == eb_ref[0]).astype(jnp.float32)
        uexp = lax.dot_general(onehot, u_ref[...], (((0,), (0,)), ((), ())),
                               preferred_element_type=jnp.float32,
                               precision=lax.Precision.HIGHEST)
        e_in = jnp.concatenate([xs, xd, ea_v, uexp], axis=1)
        h = jnp.maximum(
            jnp.dot(e_in, W1_ref[...], preferred_element_type=jnp.float32)
            + b1_ref[...], 0.0)
        h = jnp.maximum(
            jnp.dot(h, W2_ref[...], preferred_element_type=jnp.float32)
            + b2_ref[...], 0.0)
        e1 = jnp.maximum(
            jnp.dot(h, W3_ref[...], preferred_element_type=jnp.float32)
            + b3_ref[...], 0.0)
        e1_ref[...] = e1
        eao_ref[...] = e1 + ea_v
        agge_ref[...] += lax.dot_general(onehot, e1, (((1,), (0,)), ((), ())),
                                         preferred_element_type=jnp.float32,
                                         precision=lax.Precision.HIGHEST)

    H = W2.shape[0]
    return pl.pallas_call(
        body,
        grid=(G,),
        in_specs=[
            pl.BlockSpec((2, _TE, D), lambda i: (0, i, 0)),
            pl.BlockSpec((_TE, D), lambda i: (i, 0)),
            pl.BlockSpec((1, 1, _TE), lambda i: (i, 0, 0)),
            pl.BlockSpec((B, D), lambda i: (0, 0)),
            pl.BlockSpec((4 * D, H), lambda i: (0, 0)),
            pl.BlockSpec((H, H), lambda i: (0, 0)),
            pl.BlockSpec((H, D), lambda i: (0, 0)),
            pl.BlockSpec((1, H), lambda i: (0, 0)),
            pl.BlockSpec((1, H), lambda i: (0, 0)),
            pl.BlockSpec((1, D), lambda i: (0, 0)),
        ],
        out_specs=[
            pl.BlockSpec((_TE, D), lambda i: (i, 0)),
            pl.BlockSpec((_TE, D), lambda i: (i, 0)),
            pl.BlockSpec((B, D), lambda i: (0, 0)),
        ],
        out_shape=[
            jax.ShapeDtypeStruct((E, D), jnp.float32),
            jax.ShapeDtypeStruct((E, D), jnp.float32),
            jax.ShapeDtypeStruct((B, D), jnp.float32),
        ],
    )(xsxd, ea, eb3, u, W1, W2, W3, b1, b2, b3)


def _tc_node(sr, x, nb3, u, agg_e, node_params, globl_params):
    N, D = x.shape
    B = u.shape[0]
    G = N // _TN
    (Wn1, bn1), (Wn2, bn2), (Wn3, bn3) = node_params
    has_global = globl_params is not None
    if has_global:
        (Wg1, bg1), (Wg2, bg2), (Wg3, bg3) = globl_params
    H = Wn2.shape[0]

    def body(sr_ref, x_ref, nb_ref, u_ref, agge_ref,
             Wn1_ref, Wn2_ref, Wn3_ref, bn1_ref, bn2_ref, bn3_ref,
             *rest):
        if has_global:
            (Wg1_ref, Wg2_ref, Wg3_ref, bg1_ref, bg2_ref, bg3_ref,
             xo_ref, uo_ref, aggn_scr) = rest
        else:
            (xo_ref, aggn_scr) = rest
        i = pl.program_id(0)

        @pl.when(i == 0)
        def _():
            aggn_scr[...] = jnp.zeros_like(aggn_scr)

        sent = sr_ref[0]
        recv = sr_ref[1]
        x_v = x_ref[...]
        onehot = (lax.broadcasted_iota(jnp.int32, (B, _TN), 0)
                  == nb_ref[0]).astype(jnp.float32)
        unexp = lax.dot_general(onehot, u_ref[...], (((0,), (0,)), ((), ())),
                                preferred_element_type=jnp.float32,
                                precision=lax.Precision.HIGHEST)
        n_in = jnp.concatenate([x_v, recv, sent, unexp], axis=1)
        h = jnp.maximum(
            jnp.dot(n_in, Wn1_ref[...], preferred_element_type=jnp.float32)
            + bn1_ref[...], 0.0)
        h = jnp.maximum(
            jnp.dot(h, Wn2_ref[...], preferred_element_type=jnp.float32)
            + bn2_ref[...], 0.0)
        x1 = jnp.maximum(
            jnp.dot(h, Wn3_ref[...], preferred_element_type=jnp.float32)
            + bn3_ref[...], 0.0)
        xo_ref[...] = x1 + x_v
        aggn_scr[...] += lax.dot_general(onehot, x1, (((1,), (0,)), ((), ())),
                                         preferred_element_type=jnp.float32,
                                         precision=lax.Precision.HIGHEST)

        if has_global:
            @pl.when(i == G - 1)
            def _():
                u_v = u_ref[...]
                g_in = jnp.concatenate(
                    [u_v, aggn_scr[...], agge_ref[...]], axis=1)
                g = jnp.maximum(
                    jnp.dot(g_in, Wg1_ref[...],
                            preferred_element_type=jnp.float32)
                    + bg1_ref[...], 0.0)
                g = jnp.maximum(
                    jnp.dot(g, Wg2_ref[...], preferred_element_type=jnp.float32)
                    + bg2_ref[...], 0.0)
                u1 = jnp.maximum(
                    jnp.dot(g, Wg3_ref[...], preferred_element_type=jnp.float32)
                    + bg3_ref[...], 0.0)
                uo_ref[...] = u1 + u_v

    in_specs = [
        pl.BlockSpec((2, _TN, D), lambda i: (0, i, 0)),
        pl.BlockSpec((_TN, D), lambda i: (i, 0)),
        pl.BlockSpec((1, 1, _TN), lambda i: (i, 0, 0)),
        pl.BlockSpec((B, D), lambda i: (0, 0)),
        pl.BlockSpec((B, D), lambda i: (0, 0)),
        pl.BlockSpec((4 * D, H), lambda i: (0, 0)),
        pl.BlockSpec((H, H), lambda i: (0, 0)),
        pl.BlockSpec((H, D), lambda i: (0, 0)),
        pl.BlockSpec((1, H), lambda i: (0, 0)),
        pl.BlockSpec((1, H), lambda i: (0, 0)),
        pl.BlockSpec((1, D), lambda i: (0, 0)),
    ]
    args = [sr, x, nb3, u, agg_e, Wn1, Wn2, Wn3,
            bn1.reshape(1, -1), bn2.reshape(1, -1), bn3.reshape(1, -1)]
    out_specs = [pl.BlockSpec((_TN, D), lambda i: (i, 0))]
    out_shape = [jax.ShapeDtypeStruct((N, D), jnp.float32)]
    if has_global:
        in_specs += [
            pl.BlockSpec((3 * D, H), lambda i: (0, 0)),
            pl.BlockSpec((H, H), lambda i: (0, 0)),
            pl.BlockSpec((H, D), lambda i: (0, 0)),
            pl.BlockSpec((1, H), lambda i: (0, 0)),
            pl.BlockSpec((1, H), lambda i: (0, 0)),
            pl.BlockSpec((1, D), lambda i: (0, 0)),
        ]
        args += [Wg1, Wg2, Wg3,
                 bg1.reshape(1, -1), bg2.reshape(1, -1), bg3.reshape(1, -1)]
        out_specs += [pl.BlockSpec((B, D), lambda i: (0, 0))]
        out_shape += [jax.ShapeDtypeStruct((B, D), jnp.float32)]

    outs = pl.pallas_call(
        body,
        grid=(G,),
        in_specs=in_specs,
        out_specs=out_specs,
        out_shape=out_shape,
        scratch_shapes=[pltpu.VMEM((B, D), jnp.float32)],
    )(*args)
    if has_global:
        return outs[0], outs[1]
    return outs[0], u


def kernel(x, edge_index, edge_attr, u, node_batch, edge_batch, params):
    N, D = x.shape
    E = edge_attr.shape[0]
    ei3 = edge_index.astype(jnp.int32).reshape(2, E // 128, 128)
    eb3 = edge_batch.astype(jnp.int32).reshape(E // _TE, 1, _TE)
    nb3 = node_batch.astype(jnp.int32).reshape(N // _TN, 1, _TN)
    zeros_n = jnp.zeros((N, D), jnp.float32)

    for layer in params:
        (W1, b1), (W2, b2), (W3, b3) = layer['edge']
        xsxd = _sc_gather(x, ei3)
        e1, ea_new, agg_e = _tc_edge(
            xsxd, edge_attr, eb3, u, W1, W2, W3,
            b1.reshape(1, -1), b2.reshape(1, -1), b3.reshape(1, -1))
        sr = _sc_scatter(e1, ei3, zeros_n)
        x, u = _tc_node(sr, x, nb3, u, agg_e, layer['node'], layer['globl'])
        edge_attr = ea_new
    return (x, edge_attr, u)


# speedup vs baseline: 2.8313x; 1.0032x over previous
import functools

import jax
import jax.numpy as jnp
from jax import lax
from jax.experimental import pallas as pl
from jax.experimental.pallas import tpu as pltpu
from jax.experimental.pallas import tpu_sc as plsc

_NC = 2
_NS = 16
_SCH_G = 256
_SCH_S = 128
_TE = 512
_TN = 1000


def _sc_mesh():
    return plsc.VectorSubcoreMesh(core_axis_name="c", subcore_axis_name="s",
                                  num_cores=_NC, num_subcores=_NS)


def _sc_gather(x, ei3):
    N, D = x.shape
    E = ei3.shape[1] * 128
    n_sc = E // _SCH_G
    ir = _SCH_G // 128
    per_tile = n_sc // _NS
    rem = n_sc - per_tile * _NS
    assert (per_tile - 2) % 2 == 0

    @functools.partial(
        pl.kernel,
        out_type=jax.ShapeDtypeStruct((_NC, E, D), x.dtype),
        mesh=_sc_mesh(),
        scratch_types=[
            pltpu.VMEM((ir, 128), jnp.int32),
            pltpu.VMEM((ir, 128), jnp.int32),
            pltpu.VMEM((_SCH_G, D), x.dtype),
            pltpu.VMEM((_SCH_G, D), x.dtype),
            pltpu.SemaphoreType.DMA,
            pltpu.SemaphoreType.DMA,
            pltpu.SemaphoreType.DMA,
            pltpu.SemaphoreType.DMA,
        ],
    )
    def gather_kernel(x_hbm, ei_hbm, out_hbm, idx0, idx1, rows0, rows1,
                      gsem0, gsem1, ssem0, ssem1):
        idx_b = (idx0, idx1)
        rows_b = (rows0, rows1)
        gsem = (gsem0, gsem1)
        ssem = (ssem0, ssem1)
        cid = lax.axis_index("c")
        sid = lax.axis_index("s")

        def chunk(i):
            return i * _NS + sid

        def fire_gathers(b, sc):
            for j in range(ir):
                pltpu.sync_copy(ei_hbm.at[cid, sc * ir + j], idx_b[b].at[j])
            for j in range(ir):
                pltpu.async_copy(x_hbm.at[idx_b[b].at[j]],
                                 rows_b[b].at[pl.ds(j * 128, 128)], gsem[b])

        def wait_gathers(b):
            pltpu.make_async_copy(out_hbm.at[cid, pl.ds(0, _SCH_G)],
                                  rows_b[b], gsem[b]).wait()

        def fire_store(b, sc):
            pltpu.async_copy(rows_b[b],
                             out_hbm.at[cid, pl.ds(sc * _SCH_G, _SCH_G)],
                             ssem[b])

        def wait_store(b):
            pltpu.make_async_copy(rows_b[b],
                                  out_hbm.at[cid, pl.ds(0, _SCH_G)],
                                  ssem[b]).wait()

        fire_gathers(0, chunk(0))
        fire_gathers(1, chunk(1))
        wait_gathers(0)
        fire_store(0, chunk(0))

        def body(s, carry):
            for k in (0, 1):
                i = 2 * s + 2 + k
                c = chunk(i)
                wait_store(k)
                fire_gathers(k, c)
                wait_gathers(1 - k)
                fire_store(1 - k, c - _NS)
            return carry

        lax.fori_loop(0, (per_tile - 2) // 2, body, 0)
        wait_gathers(1)
        fire_store(1, chunk(per_tile - 1))
        wait_store(0)
        wait_store(1)

        def do_chunk_serial(sc):
            for j in range(ir):
                pltpu.sync_copy(ei_hbm.at[cid, sc * ir + j], idx_b[0].at[j])
            cps = [
                pltpu.async_copy(x_hbm.at[idx_b[0].at[j]],
                                 rows_b[0].at[pl.ds(j * 128, 128)], gsem[0])
                for j in range(ir)
            ]
            for cp in cps:
                cp.wait()
            pltpu.sync_copy(rows_b[0],
                            out_hbm.at[cid, pl.ds(sc * _SCH_G, _SCH_G)])

        if rem:
            @pl.when(sid < rem)
            def _():
                do_chunk_serial(per_tile * _NS + sid)

    return gather_kernel(x, ei3)


def _sc_scatter(e1, ei3, zeros_n):
    E, D = e1.shape
    N = zeros_n.shape[0]
    n_sc = E // _SCH_S
    ir = _SCH_S // 128
    per_tile = n_sc // _NS
    rem = n_sc - per_tile * _NS
    npt = (N // _NS) // 8 * 8
    ntail = N - npt * _NS

    @functools.partial(
        pl.kernel,
        out_type=jax.ShapeDtypeStruct((_NC, N, D), jnp.float32),
        mesh=_sc_mesh(),
        scratch_types=(
            [pltpu.VMEM((128,), jnp.int32) for _ in range(2 * ir)]
            + [pltpu.VMEM((_SCH_S, D), jnp.float32),
               pltpu.VMEM((_SCH_S, D), jnp.float32),
               pltpu.VMEM_SHARED((N, D), jnp.float32),
               pltpu.SemaphoreType.DMA,
               pltpu.SemaphoreType.DMA]
        ),
    )
    def scatter_kernel(e1_hbm, ei_hbm, z_hbm, out_hbm, *refs):
        idx_b = (refs[:ir], refs[ir:2 * ir])
        rows_b = (refs[2 * ir], refs[2 * ir + 1])
        accum = refs[2 * ir + 2]
        lsem = (refs[2 * ir + 3], refs[2 * ir + 4])
        cid = lax.axis_index("c")
        sid = lax.axis_index("s")
        pltpu.sync_copy(z_hbm.at[pl.ds(sid * npt, npt)],
                        accum.at[pl.ds(sid * npt, npt)])
        if ntail:
            @pl.when(sid == 0)
            def _():
                pltpu.sync_copy(z_hbm.at[pl.ds(npt * _NS, ntail)],
                                accum.at[pl.ds(npt * _NS, ntail)])
        plsc.subcore_barrier()

        last = per_tile - 1

        def fire(b, sc):
            pltpu.async_copy(e1_hbm.at[pl.ds(sc * _SCH_S, _SCH_S)],
                             rows_b[b], lsem[b])

        def proc(b, sc):
            for j in range(ir):
                pltpu.sync_copy(ei_hbm.at[cid, sc * ir + j], idx_b[b][j])
            pltpu.make_async_copy(e1_hbm.at[pl.ds(sc * _SCH_S, _SCH_S)],
                                  rows_b[b], lsem[b]).wait()
            for j in range(ir):
                pltpu.sync_copy(rows_b[b].at[pl.ds(j * 128, 128)],
                                accum.at[idx_b[b][j]], add=True)

        def chunk(i):
            return i * _NS + sid

        fire(0, chunk(0))
        fire(1, chunk(1))

        def body(s, carry):
            for k in (0, 1):
                i = 2 * s + k
                proc(k, chunk(i))
                nxt = jnp.minimum(i + 2, last)
                fire(k, chunk(nxt))
            return carry

        lax.fori_loop(0, per_tile // 2, body, 0)
        for b in (0, 1):
            pltpu.make_async_copy(e1_hbm.at[pl.ds(0, _SCH_S)],
                                  rows_b[b], lsem[b]).wait()

        def do_chunk(sc):
            for j in range(ir):
                pltpu.sync_copy(ei_hbm.at[cid, sc * ir + j], idx_b[0][j])
            pltpu.sync_copy(e1_hbm.at[pl.ds(sc * _SCH_S, _SCH_S)], rows_b[0])
            for j in range(ir):
                pltpu.sync_copy(rows_b[0].at[pl.ds(j * 128, 128)],
                                accum.at[idx_b[0][j]], add=True)

        if rem:
            @pl.when(sid < rem)
            def _():
                do_chunk(per_tile * _NS + sid)
        plsc.subcore_barrier()
        pltpu.sync_copy(accum.at[pl.ds(sid * npt, npt)],
                        out_hbm.at[cid, pl.ds(sid * npt, npt)])
        if ntail:
            @pl.when(sid == 0)
            def _():
                pltpu.sync_copy(accum.at[pl.ds(npt * _NS, ntail)],
                                out_hbm.at[cid, pl.ds(npt * _NS, ntail)])

    return scatter_kernel(e1, ei3, zeros_n)


def _tc_edge(xsxd, ea, eb3, u, W1, W2, W3, b1, b2, b3):
    E, D = ea.shape
    B = u.shape[0]
    G = E // _TE

    def body(xsxd_ref, ea_ref, eb_ref, u_ref, W1_ref, W2_ref, W3_ref,
             b1_ref, b2_ref, b3_ref, e1_ref, eao_ref, agge_ref):
        i = pl.program_id(0)

        @pl.when(i == 0)
        def _():
            agge_ref[...] = jnp.zeros_like(agge_ref)

        xs = xsxd_ref[0]
        xd = xsxd_ref[1]
        ea_v = ea_ref[...]
        onehot = (lax.broadcasted_iota(jnp.int32, (B, _TE), 0)
                  == eb_ref[0]).astype(jnp.float32)
        uexp = lax.dot_general(onehot, u_ref[...], (((0,), (0,)), ((), ())),
                               preferred_element_type=jnp.float32,
                               precision=lax.Precision.HIGHEST)
        e_in = jnp.concatenate([xs, xd, ea_v, uexp], axis=1)
        h = jnp.maximum(
            jnp.dot(e_in, W1_ref[...], preferred_element_type=jnp.float32)
            + b1_ref[...], 0.0)
        h = jnp.maximum(
            jnp.dot(h, W2_ref[...], preferred_element_type=jnp.float32)
            + b2_ref[...], 0.0)
        e1 = jnp.maximum(
            jnp.dot(h, W3_ref[...], preferred_element_type=jnp.float32)
            + b3_ref[...], 0.0)
        e1_ref[...] = e1
        eao_ref[...] = e1 + ea_v
        agge_ref[...] += lax.dot_general(onehot, e1, (((1,), (0,)), ((), ())),
                                         preferred_element_type=jnp.float32,
                                         precision=lax.Precision.HIGHEST)

    H = W2.shape[0]
    return pl.pallas_call(
        body,
        grid=(G,),
        in_specs=[
            pl.BlockSpec((2, _TE, D), lambda i: (0, i, 0)),
            pl.BlockSpec((_TE, D), lambda i: (i, 0)),
            pl.BlockSpec((1, 1, _TE), lambda i: (i, 0, 0)),
            pl.BlockSpec((B, D), lambda i: (0, 0)),
            pl.BlockSpec((4 * D, H), lambda i: (0, 0)),
            pl.BlockSpec((H, H), lambda i: (0, 0)),
            pl.BlockSpec((H, D), lambda i: (0, 0)),
            pl.BlockSpec((1, H), lambda i: (0, 0)),
            pl.BlockSpec((1, H), lambda i: (0, 0)),
            pl.BlockSpec((1, D), lambda i: (0, 0)),
        ],
        out_specs=[
            pl.BlockSpec((_TE, D), lambda i: (i, 0)),
            pl.BlockSpec((_TE, D), lambda i: (i, 0)),
            pl.BlockSpec((B, D), lambda i: (0, 0)),
        ],
        out_shape=[
            jax.ShapeDtypeStruct((E, D), jnp.float32),
            jax.ShapeDtypeStruct((E, D), jnp.float32),
            jax.ShapeDtypeStruct((B, D), jnp.float32),
        ],
    )(xsxd, ea, eb3, u, W1, W2, W3, b1, b2, b3)


def _tc_node(sr, x, nb3, u, agg_e, node_params, globl_params):
    N, D = x.shape
    B = u.shape[0]
    G = N // _TN
    (Wn1, bn1), (Wn2, bn2), (Wn3, bn3) = node_params
    has_global = globl_params is not None
    if has_global:
        (Wg1, bg1), (Wg2, bg2), (Wg3, bg3) = globl_params
    H = Wn2.shape[0]

    def body(sr_ref, x_ref, nb_ref, u_ref, agge_ref,
             Wn1_ref, Wn2_ref, Wn3_ref, bn1_ref, bn2_ref, bn3_ref,
             *rest):
        if has_global:
            (Wg1_ref, Wg2_ref, Wg3_ref, bg1_ref, bg2_ref, bg3_ref,
             xo_ref, uo_ref, aggn_scr) = rest
        else:
            (xo_ref, aggn_scr) = rest
        i = pl.program_id(0)

        @pl.when(i == 0)
        def _():
            aggn_scr[...] = jnp.zeros_like(aggn_scr)

        sent = sr_ref[0]
        recv = sr_ref[1]
        x_v = x_ref[...]
        onehot = (lax.broadcasted_iota(jnp.int32, (B, _TN), 0)
                  == nb_ref[0]).astype(jnp.float32)
        unexp = lax.dot_general(onehot, u_ref[...], (((0,), (0,)), ((), ())),
                                preferred_element_type=jnp.float32,
                                precision=lax.Precision.HIGHEST)
        n_in = jnp.concatenate([x_v, recv, sent, unexp], axis=1)
        h = jnp.maximum(
            jnp.dot(n_in, Wn1_ref[...], preferred_element_type=jnp.float32)
            + bn1_ref[...], 0.0)
        h = jnp.maximum(
            jnp.dot(h, Wn2_ref[...], preferred_element_type=jnp.float32)
            + bn2_ref[...], 0.0)
        x1 = jnp.maximum(
            jnp.dot(h, Wn3_ref[...], preferred_element_type=jnp.float32)
            + bn3_ref[...], 0.0)
        xo_ref[...] = x1 + x_v
        aggn_scr[...] += lax.dot_general(onehot, x1, (((1,), (0,)), ((), ())),
                                         preferred_element_type=jnp.float32,
                                         precision=lax.Precision.HIGHEST)

        if has_global:
            @pl.when(i == G - 1)
            def _():
                u_v = u_ref[...]
                g_in = jnp.concatenate(
                    [u_v, aggn_scr[...], agge_ref[...]], axis=1)
                g = jnp.maximum(
                    jnp.dot(g_in, Wg1_ref[...],
                            preferred_element_type=jnp.float32)
                    + bg1_ref[...], 0.0)
                g = jnp.maximum(
                    jnp.dot(g, Wg2_ref[...], preferred_element_type=jnp.float32)
                    + bg2_ref[...], 0.0)
                u1 = jnp.maximum(
                    jnp.dot(g, Wg3_ref[...], preferred_element_type=jnp.float32)
                    + bg3_ref[...], 0.0)
                uo_ref[...] = u1 + u_v

    in_specs = [
        pl.BlockSpec((2, _TN, D), lambda i: (0, i, 0)),
        pl.BlockSpec((_TN, D), lambda i: (i, 0)),
        pl.BlockSpec((1, 1, _TN), lambda i: (i, 0, 0)),
        pl.BlockSpec((B, D), lambda i: (0, 0)),
        pl.BlockSpec((B, D), lambda i: (0, 0)),
        pl.BlockSpec((4 * D, H), lambda i: (0, 0)),
        pl.BlockSpec((H, H), lambda i: (0, 0)),
        pl.BlockSpec((H, D), lambda i: (0, 0)),
        pl.BlockSpec((1, H), lambda i: (0, 0)),
        pl.BlockSpec((1, H), lambda i: (0, 0)),
        pl.BlockSpec((1, D), lambda i: (0, 0)),
    ]
    args = [sr, x, nb3, u, agg_e, Wn1, Wn2, Wn3,
            bn1.reshape(1, -1), bn2.reshape(1, -1), bn3.reshape(1, -1)]
    out_specs = [pl.BlockSpec((_TN, D), lambda i: (i, 0))]
    out_shape = [jax.ShapeDtypeStruct((N, D), jnp.float32)]
    if has_global:
        in_specs += [
            pl.BlockSpec((3 * D, H), lambda i: (0, 0)),
            pl.BlockSpec((H, H), lambda i: (0, 0)),
            pl.BlockSpec((H, D), lambda i: (0, 0)),
            pl.BlockSpec((1, H), lambda i: (0, 0)),
            pl.BlockSpec((1, H), lambda i: (0, 0)),
            pl.BlockSpec((1, D), lambda i: (0, 0)),
        ]
        args += [Wg1, Wg2, Wg3,
                 bg1.reshape(1, -1), bg2.reshape(1, -1), bg3.reshape(1, -1)]
        out_specs += [pl.BlockSpec((B, D), lambda i: (0, 0))]
        out_shape += [jax.ShapeDtypeStruct((B, D), jnp.float32)]

    outs = pl.pallas_call(
        body,
        grid=(G,),
        in_specs=in_specs,
        out_specs=out_specs,
        out_shape=out_shape,
        scratch_shapes=[pltpu.VMEM((B, D), jnp.float32)],
    )(*args)
    if has_global:
        return outs[0], outs[1]
    return outs[0], u


def kernel(x, edge_index, edge_attr, u, node_batch, edge_batch, params):
    N, D = x.shape
    E = edge_attr.shape[0]
    ei3 = edge_index.astype(jnp.int32).reshape(2, E // 128, 128)
    eb3 = edge_batch.astype(jnp.int32).reshape(E // _TE, 1, _TE)
    nb3 = node_batch.astype(jnp.int32).reshape(N // _TN, 1, _TN)
    zeros_n = jnp.zeros((N, D), jnp.float32)

    for layer in params:
        (W1, b1), (W2, b2), (W3, b3) = layer['edge']
        xsxd = _sc_gather(x, ei3)
        e1, ea_new, agg_e = _tc_edge(
            xsxd, edge_attr, eb3, u, W1, W2, W3,
            b1.reshape(1, -1), b2.reshape(1, -1), b3.reshape(1, -1))
        sr = _sc_scatter(e1, ei3, zeros_n)
        x, u = _tc_node(sr, x, nb3, u, agg_e, layer['node'], layer['globl'])
        edge_attr = ea_new
    return (x, edge_attr, u)
